# baseline (speedup 1.0000x reference)
import functools

import jax
import jax.numpy as jnp
from jax import lax
from jax.experimental import pallas as pl
from jax.experimental.pallas import tpu as pltpu
from jax.experimental.pallas import tpu_sc as plsc

BATCH = 8
IMG = 224
PS = 16
GRID = IMG // PS
NPATCH = GRID * GRID
SEQ = NPATCH + 1
SP = 256
HID = 768
HEADS = 12
DKV = 64
DFF = 1024
E = 16
DEPTH = 4
NCLS = 1000
NTOK = BATCH * SEQ
NP = BATCH * SP
CAP = int(2.0 * NTOK / E)
CAPP = 208
ECAP = E * CAPP
XROWS = ECAP + CAPP
NW = 32
TPW = NP // NW

_f32 = jnp.float32


def _layernorm(x, s, b):
    m = jnp.mean(x, -1, keepdims=True)
    c = x - m
    v = jnp.mean(c * c, -1, keepdims=True)
    return c * lax.rsqrt(v + 1e-6) * s + b


def _embed_body(p_ref, w_ref, b_ref, pos_ref, o_ref):
    o_ref[...] = (
        jnp.dot(p_ref[...], w_ref[...], preferred_element_type=_f32)
        + b_ref[...] + pos_ref[...]
    )


def _lnqkv_body(h_ref, s_ref, b_ref, wq_ref, bq_ref, wk_ref, bk_ref,
                wv_ref, bv_ref, q_ref, k_ref, v_ref):
    hn = _layernorm(h_ref[...], s_ref[...], b_ref[...])
    q_ref[...] = jnp.dot(hn, wq_ref[...], preferred_element_type=_f32) + bq_ref[...]
    k_ref[...] = jnp.dot(hn, wk_ref[...], preferred_element_type=_f32) + bk_ref[...]
    v_ref[...] = jnp.dot(hn, wv_ref[...], preferred_element_type=_f32) + bv_ref[...]


def _attn_body(q_ref, k_ref, v_ref, o_ref):
    q = q_ref[...][0, 0]
    s = lax.dot_general(q, k_ref[...][0, 0], (((1,), (1,)), ((), ())),
                        preferred_element_type=_f32) * 0.125
    cols = lax.broadcasted_iota(jnp.int32, (SP, SP), 1)
    s = jnp.where(cols < SEQ, s, -1e30)
    m = jnp.max(s, -1, keepdims=True)
    e = jnp.exp(s - m)
    a = e / jnp.sum(e, -1, keepdims=True)
    o_ref[...] = jnp.dot(a, v_ref[...][0, 0],
                         preferred_element_type=_f32)[None, None]


def _proj_body(h_ref, ao_ref, wo_ref, bo_ref, s2_ref, b2_ref, gw_ref,
               h2_ref, hn2_ref, probs_ref):
    h2 = h_ref[...] + jnp.dot(ao_ref[...], wo_ref[...],
                              preferred_element_type=_f32) + bo_ref[...]
    h2_ref[...] = h2
    hn2 = _layernorm(h2, s2_ref[...], b2_ref[...])
    hn2_ref[...] = hn2
    logits = jnp.dot(hn2, gw_ref[...], preferred_element_type=_f32)
    mx = jnp.max(logits, -1, keepdims=True)
    ex = jnp.exp(logits - mx)
    probs_ref[...] = ex / jnp.sum(ex, -1, keepdims=True)


def _router_body(probs_ref, slot1_ref, slot2_ref, g1_ref, g2_ref):
    p = probs_ref[...]
    rows = lax.broadcasted_iota(jnp.int32, (NP, E), 0)
    cols = lax.broadcasted_iota(jnp.int32, (NP, E), 1)
    tokvalid = (rows % SP) < SEQ
    pmax1 = jnp.max(p, -1, keepdims=True)
    idx1 = jnp.min(jnp.where(p >= pmax1, cols, E), -1, keepdims=True)
    m1 = ((cols == idx1) & tokvalid).astype(_f32)
    p2m = p * (1.0 - (cols == idx1).astype(_f32))
    pmax2 = jnp.max(p2m, -1, keepdims=True)
    idx2 = jnp.min(jnp.where(p2m >= pmax2, cols, E), -1, keepdims=True)
    m2 = ((cols == idx2) & tokvalid).astype(_f32)

    tri = (lax.broadcasted_iota(jnp.int32, (SP, SP), 0)
           > lax.broadcasted_iota(jnp.int32, (SP, SP), 1)).astype(_f32)

    def exc_cumsum(m):
        outs = []
        carry = jnp.zeros((1, E), _f32)
        for b in range(BATCH):
            blk = lax.slice(m, (b * SP, 0), ((b + 1) * SP, E))
            outs.append(jnp.dot(tri, blk, preferred_element_type=_f32) + carry)
            carry = carry + jnp.sum(blk, 0, keepdims=True)
        return jnp.concatenate(outs, 0), carry

    loc1, cnt1 = exc_cumsum(m1)
    loc2, _ = exc_cumsum(m2)
    loc2 = loc2 + cnt1
    m1c = m1 * (loc1 < CAP)
    m2c = m2 * (loc2 < CAP)
    p1 = jnp.sum(loc1 * m1c, -1, keepdims=True).astype(jnp.int32)
    p2 = jnp.sum(loc2 * m2c, -1, keepdims=True).astype(jnp.int32)
    g1 = jnp.sum(p * m1c, -1, keepdims=True)
    g2 = jnp.sum(p * m2c, -1, keepdims=True)
    den = g1 + g2 + 1e-9
    v1 = jnp.sum(m1c, -1, keepdims=True) > 0.0
    v2 = jnp.sum(m2c, -1, keepdims=True) > 0.0
    slot1_ref[...] = jnp.where(v1, idx1 * CAPP + p1, ECAP)
    slot2_ref[...] = jnp.where(v2, idx2 * CAPP + p2, ECAP)
    g1_ref[...] = jnp.where(v1, g1 / den, 0.0)
    g2_ref[...] = jnp.where(v2, g2 / den, 0.0)


def _ffn_body(x_ref, w1_ref, b1_ref, w2_ref, b2_ref, o_ref):
    h = jnp.dot(x_ref[...], w1_ref[...][0], preferred_element_type=_f32) + b1_ref[...][0]
    h = jax.nn.gelu(h)
    o_ref[...] = jnp.dot(h, w2_ref[...][0], preferred_element_type=_f32) + b2_ref[...][0]


def _combine_body(h2_ref, r1_ref, r2_ref, slot1_ref, slot2_ref,
                  g1_ref, g2_ref, out_ref):
    v1 = slot1_ref[...] < ECAP
    v2 = slot2_ref[...] < ECAP
    out_ref[...] = (h2_ref[...]
                    + jnp.where(v1, g1_ref[...] * r1_ref[...], 0.0)
                    + jnp.where(v2, g2_ref[...] * r2_ref[...], 0.0))


def _pool_body(h_ref, s_ref, b_ref, out_ref):
    hn = _layernorm(h_ref[...], s_ref[...], b_ref[...])
    rows = lax.broadcasted_iota(jnp.int32, (SP, 1), 0)
    out_ref[...] = jnp.sum(jnp.where(rows < SEQ, hn, 0.0), 0,
                           keepdims=True)[None] * (1.0 / SEQ)


def _head_body(p_ref, w_ref, b_ref, o_ref):
    o_ref[...] = jnp.dot(p_ref[...], w_ref[...], preferred_element_type=_f32) + b_ref[...]


def _row_spec(n):
    return pl.BlockSpec((n, HID), lambda i: (i, 0))


def _full(shape):
    return pl.BlockSpec(shape, lambda *a: tuple(0 for _ in shape))


def _scmesh():
    return plsc.VectorSubcoreMesh(core_axis_name="c", subcore_axis_name="s")


def _sc_dispatch(tok, slot1, slot2):
    @functools.partial(
        pl.kernel,
        out_type=jax.ShapeDtypeStruct((XROWS, HID), _f32),
        mesh=_scmesh(),
        scratch_types=[
            pltpu.VMEM((TPW,), jnp.int32),
            pltpu.VMEM((TPW,), jnp.int32),
            pltpu.VMEM((TPW, HID), _f32),
            pltpu.SemaphoreType.DMA,
            pltpu.SemaphoreType.DMA,
        ],
    )
    def k(tok_hbm, s1_hbm, s2_hbm, out_hbm, i1_v, i2_v, rows_v, sem1, sem2):
        wid = lax.axis_index("s") * 2 + lax.axis_index("c")
        base = wid * TPW
        c1 = pltpu.async_copy(s1_hbm.at[pl.ds(base, TPW)], i1_v, sem1)
        c2 = pltpu.async_copy(s2_hbm.at[pl.ds(base, TPW)], i2_v, sem2)
        pltpu.sync_copy(tok_hbm.at[pl.ds(base, TPW)], rows_v)
        c1.wait()
        c2.wait()
        s1 = pltpu.async_copy(rows_v, out_hbm.at[i1_v], sem1)
        s2 = pltpu.async_copy(rows_v, out_hbm.at[i2_v], sem2)
        s1.wait()
        s2.wait()

    return k(tok, slot1, slot2)


def _sc_combine(eo, slot1, slot2):
    @functools.partial(
        pl.kernel,
        out_type=(jax.ShapeDtypeStruct((NP, HID), _f32),
                  jax.ShapeDtypeStruct((NP, HID), _f32)),
        mesh=_scmesh(),
        scratch_types=[
            pltpu.VMEM((TPW,), jnp.int32),
            pltpu.VMEM((TPW,), jnp.int32),
            pltpu.VMEM((TPW, HID), _f32),
            pltpu.VMEM((TPW, HID), _f32),
            pltpu.SemaphoreType.DMA,
            pltpu.SemaphoreType.DMA,
        ],
    )
    def k(eo_hbm, s1_hbm, s2_hbm, r1_hbm, r2_hbm, i1_v, i2_v, r1_v, r2_v,
          sem1, sem2):
        wid = lax.axis_index("s") * 2 + lax.axis_index("c")
        base = wid * TPW
        c1 = pltpu.async_copy(s1_hbm.at[pl.ds(base, TPW)], i1_v, sem1)
        c2 = pltpu.async_copy(s2_hbm.at[pl.ds(base, TPW)], i2_v, sem2)
        c1.wait()
        c2.wait()
        g1 = pltpu.async_copy(eo_hbm.at[i1_v], r1_v, sem1)
        g2 = pltpu.async_copy(eo_hbm.at[i2_v], r2_v, sem2)
        g1.wait()
        w1 = pltpu.async_copy(r1_v, r1_hbm.at[pl.ds(base, TPW)], sem1)
        g2.wait()
        w2 = pltpu.async_copy(r2_v, r2_hbm.at[pl.ds(base, TPW)], sem2)
        w1.wait()
        w2.wait()

    return k(eo, slot1, slot2)


def _embed(patches, wpatch, bpatch, posb):
    return pl.pallas_call(
        _embed_body,
        grid=(1,),
        in_specs=[_full((BATCH * NPATCH, HID)), _full((HID, HID)),
                  _full((1, HID)), _full((BATCH * NPATCH, HID))],
        out_specs=_full((BATCH * NPATCH, HID)),
        out_shape=jax.ShapeDtypeStruct((BATCH * NPATCH, HID), _f32),
    )(patches, wpatch, bpatch, posb)


def _lnqkv(h, s, b, wq, bq, wk, bk, wv, bv):
    return pl.pallas_call(
        _lnqkv_body,
        grid=(BATCH,),
        in_specs=[_row_spec(SP), _full((1, HID)), _full((1, HID)),
                  _full((HID, HID)), _full((1, HID)),
                  _full((HID, HID)), _full((1, HID)),
                  _full((HID, HID)), _full((1, HID))],
        out_specs=[_row_spec(SP)] * 3,
        out_shape=[jax.ShapeDtypeStruct((NP, HID), _f32)] * 3,
    )(h, s, b, wq, bq, wk, bk, wv, bv)


def _attn(q, k, v):
    spec = pl.BlockSpec((1, 1, SP, DKV), lambda n, h: (n, h, 0, 0))
    return pl.pallas_call(
        _attn_body,
        grid=(BATCH, HEADS),
        in_specs=[spec, spec, spec],
        out_specs=spec,
        out_shape=jax.ShapeDtypeStruct((BATCH, HEADS, SP, DKV), _f32),
    )(q, k, v)


def _proj(h, ao, wo, bo, s2, b2, gw):
    return pl.pallas_call(
        _proj_body,
        grid=(BATCH,),
        in_specs=[_row_spec(SP), _row_spec(SP), _full((HID, HID)),
                  _full((1, HID)), _full((1, HID)), _full((1, HID)),
                  _full((HID, E))],
        out_specs=[_row_spec(SP), _row_spec(SP),
                   pl.BlockSpec((SP, E), lambda i: (i, 0))],
        out_shape=[jax.ShapeDtypeStruct((NP, HID), _f32),
                   jax.ShapeDtypeStruct((NP, HID), _f32),
                   jax.ShapeDtypeStruct((NP, E), _f32)],
    )(h, ao, wo, bo, s2, b2, gw)


def _router(probs):
    one = pl.BlockSpec((NP, 1), lambda: (0, 0))
    return pl.pallas_call(
        _router_body,
        grid=(),
        in_specs=[pl.BlockSpec((NP, E), lambda: (0, 0))],
        out_specs=[one, one, one, one],
        out_shape=[jax.ShapeDtypeStruct((NP, 1), jnp.int32),
                   jax.ShapeDtypeStruct((NP, 1), jnp.int32),
                   jax.ShapeDtypeStruct((NP, 1), _f32),
                   jax.ShapeDtypeStruct((NP, 1), _f32)],
    )(probs)


def _ffn(xd, w1, b1, w2, b2):
    return pl.pallas_call(
        _ffn_body,
        grid=(E,),
        in_specs=[pl.BlockSpec((CAPP, HID), lambda e: (e, 0)),
                  pl.BlockSpec((1, HID, DFF), lambda e: (e, 0, 0)),
                  pl.BlockSpec((1, 1, DFF), lambda e: (e, 0, 0)),
                  pl.BlockSpec((1, DFF, HID), lambda e: (e, 0, 0)),
                  pl.BlockSpec((1, 1, HID), lambda e: (e, 0, 0))],
        out_specs=pl.BlockSpec((CAPP, HID), lambda e: (e, 0)),
        out_shape=jax.ShapeDtypeStruct((XROWS, HID), _f32),
    )(xd, w1, b1, w2, b2)


def _combine(h2, r1, r2, slot1, slot2, g1, g2):
    one = pl.BlockSpec((SP, 1), lambda i: (i, 0))
    return pl.pallas_call(
        _combine_body,
        grid=(BATCH,),
        in_specs=[_row_spec(SP), _row_spec(SP), _row_spec(SP),
                  one, one, one, one],
        out_specs=_row_spec(SP),
        out_shape=jax.ShapeDtypeStruct((NP, HID), _f32),
    )(h2, r1, r2, slot1, slot2, g1, g2)


def _pool(h, s, b):
    return pl.pallas_call(
        _pool_body,
        grid=(BATCH,),
        in_specs=[_row_spec(SP), _full((1, HID)), _full((1, HID))],
        out_specs=pl.BlockSpec((1, 1, HID), lambda i: (i, 0, 0)),
        out_shape=jax.ShapeDtypeStruct((BATCH, 1, HID), _f32),
    )(h, s, b)


def _head(pooled, wc, bc):
    return pl.pallas_call(
        _head_body,
        grid=(1,),
        in_specs=[_full((BATCH, HID)), _full((HID, NCLS)), _full((1, NCLS))],
        out_specs=_full((BATCH, NCLS)),
        out_shape=jax.ShapeDtypeStruct((BATCH, NCLS), _f32),
    )(pooled, wc, bc)


def kernel(x, Wpatch, bpatch, cls_tok, pos_emb, ln1_s, ln1_b, ln2_s, ln2_b,
           Wq, bq, Wk, bk, Wv, bv, Wo, bo, gate_w, W1, b1, W2, b2,
           lnf_s, lnf_b, Wc, bc):
    patches = x.reshape(BATCH, 3, GRID, PS, GRID, PS)
    patches = patches.transpose(0, 2, 4, 1, 3, 5).reshape(BATCH * NPATCH, HID)
    posb = jnp.broadcast_to(pos_emb[:, 1:SEQ], (BATCH, NPATCH, HID))
    emb = _embed(patches, Wpatch, bpatch.reshape(1, HID),
                 posb.reshape(BATCH * NPATCH, HID))
    cls_row = jnp.broadcast_to(cls_tok + pos_emb[:, :1], (BATCH, 1, HID))
    h = jnp.concatenate(
        [cls_row, emb.reshape(BATCH, NPATCH, HID),
         jnp.zeros((BATCH, SP - SEQ, HID), _f32)], axis=1).reshape(NP, HID)

    for i in range(DEPTH):
        q, k, v = _lnqkv(h, ln1_s[i].reshape(1, HID), ln1_b[i].reshape(1, HID),
                         Wq, bq.reshape(1, HID), Wk, bk.reshape(1, HID),
                         Wv, bv.reshape(1, HID))
        q4 = q.reshape(BATCH, SP, HEADS, DKV).transpose(0, 2, 1, 3)
        k4 = k.reshape(BATCH, SP, HEADS, DKV).transpose(0, 2, 1, 3)
        v4 = v.reshape(BATCH, SP, HEADS, DKV).transpose(0, 2, 1, 3)
        ao = _attn(q4, k4, v4).transpose(0, 2, 1, 3).reshape(NP, HID)
        h2, hn2, probs = _proj(h, ao, Wo, bo.reshape(1, HID),
                               ln2_s[i].reshape(1, HID),
                               ln2_b[i].reshape(1, HID), gate_w)
        slot1, slot2, g1, g2 = _router(probs)
        s1f = slot1.reshape(NP)
        s2f = slot2.reshape(NP)
        xd = _sc_dispatch(hn2, s1f, s2f)
        eo = _ffn(xd, W1, b1.reshape(E, 1, DFF), W2, b2.reshape(E, 1, HID))
        r1, r2 = _sc_combine(eo, s1f, s2f)
        h = _combine(h2, r1, r2, slot1, slot2, g1, g2)

    pooled = _pool(h, lnf_s.reshape(1, HID), lnf_b.reshape(1, HID))
    return _head(pooled.reshape(BATCH, HID), Wc, bc.reshape(1, NCLS))


# speedup vs baseline: 1.3852x; 1.3852x over previous
import functools

import jax
import jax.numpy as jnp
from jax import lax
from jax.experimental import pallas as pl
from jax.experimental.pallas import tpu as pltpu
from jax.experimental.pallas import tpu_sc as plsc

BATCH = 8
IMG = 224
PS = 16
GRID = IMG // PS
NPATCH = GRID * GRID
SEQ = NPATCH + 1
SP = 256
HID = 768
HEADS = 12
DKV = 64
DFF = 1024
E = 16
DEPTH = 4
NCLS = 1000
NTOK = BATCH * SEQ
NP = BATCH * SP
CAP = int(2.0 * NTOK / E)
CAPP = 208
ECAP = E * CAPP
XROWS = ECAP + CAPP
NW = 32
TPW = NP // NW

_f32 = jnp.float32


def _layernorm(x, s, b):
    m = jnp.mean(x, -1, keepdims=True)
    c = x - m
    v = jnp.mean(c * c, -1, keepdims=True)
    return c * lax.rsqrt(v + 1e-6) * s + b


def _embed_body(p_ref, w_ref, b_ref, pos_ref, o_ref):
    o_ref[...] = (
        jnp.dot(p_ref[...], w_ref[...], preferred_element_type=_f32)
        + b_ref[...] + pos_ref[...]
    )


def _block_body(h_ref, s1_ref, b1_ref, wq4_ref, bq4_ref, wk4_ref, bk4_ref,
                wv4_ref, bv4_ref, wo_ref, bo_ref, s2_ref, b2_ref, gw_ref,
                h2_ref, hn2_ref, probs_ref):
    h = h_ref[...]
    hn = _layernorm(h, s1_ref[...], b1_ref[...])
    cols = lax.broadcasted_iota(jnp.int32, (SP, SP), 1)
    outs = []
    for hd in range(HEADS):
        qh = jnp.dot(hn, wq4_ref[hd], preferred_element_type=_f32) + bq4_ref[...][hd]
        kh = jnp.dot(hn, wk4_ref[hd], preferred_element_type=_f32) + bk4_ref[...][hd]
        vh = jnp.dot(hn, wv4_ref[hd], preferred_element_type=_f32) + bv4_ref[...][hd]
        s = lax.dot_general(qh, kh, (((1,), (1,)), ((), ())),
                            preferred_element_type=_f32) * 0.125
        s = jnp.where(cols < SEQ, s, -1e30)
        m = jnp.max(s, -1, keepdims=True)
        e = jnp.exp(s - m)
        a = e / jnp.sum(e, -1, keepdims=True)
        outs.append(jnp.dot(a, vh, preferred_element_type=_f32))
    o = jnp.concatenate(outs, -1)
    h2 = h + jnp.dot(o, wo_ref[...], preferred_element_type=_f32) + bo_ref[...]
    h2_ref[...] = h2
    hn2 = _layernorm(h2, s2_ref[...], b2_ref[...])
    hn2_ref[...] = hn2
    logits = jnp.dot(hn2, gw_ref[...], preferred_element_type=_f32)
    mx = jnp.max(logits, -1, keepdims=True)
    ex = jnp.exp(logits - mx)
    probs_ref[...] = ex / jnp.sum(ex, -1, keepdims=True)


def _router_body(probs_ref, slot1_ref, slot2_ref, g1_ref, g2_ref):
    p = probs_ref[...]
    rows = lax.broadcasted_iota(jnp.int32, (NP, E), 0)
    cols = lax.broadcasted_iota(jnp.int32, (NP, E), 1)
    tokvalid = (rows % SP) < SEQ
    pmax1 = jnp.max(p, -1, keepdims=True)
    idx1 = jnp.min(jnp.where(p >= pmax1, cols, E), -1, keepdims=True)
    m1 = ((cols == idx1) & tokvalid).astype(_f32)
    p2m = p * (1.0 - (cols == idx1).astype(_f32))
    pmax2 = jnp.max(p2m, -1, keepdims=True)
    idx2 = jnp.min(jnp.where(p2m >= pmax2, cols, E), -1, keepdims=True)
    m2 = ((cols == idx2) & tokvalid).astype(_f32)

    tri = (lax.broadcasted_iota(jnp.int32, (SP, SP), 0)
           > lax.broadcasted_iota(jnp.int32, (SP, SP), 1)).astype(_f32)

    def exc_cumsum(m):
        outs = []
        carry = jnp.zeros((1, E), _f32)
        for b in range(BATCH):
            blk = lax.slice(m, (b * SP, 0), ((b + 1) * SP, E))
            outs.append(jnp.dot(tri, blk, preferred_element_type=_f32) + carry)
            carry = carry + jnp.sum(blk, 0, keepdims=True)
        return jnp.concatenate(outs, 0), carry

    loc1, cnt1 = exc_cumsum(m1)
    loc2, _ = exc_cumsum(m2)
    loc2 = loc2 + cnt1
    m1c = m1 * (loc1 < CAP)
    m2c = m2 * (loc2 < CAP)
    p1 = jnp.sum(loc1 * m1c, -1, keepdims=True).astype(jnp.int32)
    p2 = jnp.sum(loc2 * m2c, -1, keepdims=True).astype(jnp.int32)
    g1 = jnp.sum(p * m1c, -1, keepdims=True)
    g2 = jnp.sum(p * m2c, -1, keepdims=True)
    den = g1 + g2 + 1e-9
    v1 = jnp.sum(m1c, -1, keepdims=True) > 0.0
    v2 = jnp.sum(m2c, -1, keepdims=True) > 0.0
    slot1_ref[...] = jnp.where(v1, idx1 * CAPP + p1, ECAP)
    slot2_ref[...] = jnp.where(v2, idx2 * CAPP + p2, ECAP)
    g1_ref[...] = jnp.where(v1, g1 / den, 0.0)
    g2_ref[...] = jnp.where(v2, g2 / den, 0.0)


def _ffn_body(x_ref, w1_ref, b1_ref, w2_ref, b2_ref, o_ref):
    h = jnp.dot(x_ref[...], w1_ref[...][0], preferred_element_type=_f32) + b1_ref[...][0]
    h = jax.nn.gelu(h)
    o_ref[...] = jnp.dot(h, w2_ref[...][0], preferred_element_type=_f32) + b2_ref[...][0]


def _combine_body(h2_ref, r1_ref, r2_ref, slot1_ref, slot2_ref,
                  g1_ref, g2_ref, out_ref):
    v1 = slot1_ref[...] < ECAP
    v2 = slot2_ref[...] < ECAP
    out_ref[...] = (h2_ref[...]
                    + jnp.where(v1, g1_ref[...] * r1_ref[...], 0.0)
                    + jnp.where(v2, g2_ref[...] * r2_ref[...], 0.0))


def _pool_body(h_ref, s_ref, b_ref, out_ref):
    hn = _layernorm(h_ref[...], s_ref[...], b_ref[...])
    rows = lax.broadcasted_iota(jnp.int32, (SP, 1), 0)
    out_ref[...] = jnp.sum(jnp.where(rows < SEQ, hn, 0.0), 0,
                           keepdims=True)[None] * (1.0 / SEQ)


def _head_body(p_ref, w_ref, b_ref, o_ref):
    o_ref[...] = jnp.dot(p_ref[...], w_ref[...], preferred_element_type=_f32) + b_ref[...]


def _row_spec(n):
    return pl.BlockSpec((n, HID), lambda i: (i, 0))


def _full(shape):
    return pl.BlockSpec(shape, lambda *a: tuple(0 for _ in shape))


def _scmesh():
    return plsc.VectorSubcoreMesh(core_axis_name="c", subcore_axis_name="s")


def _sc_dispatch(tok, slot1, slot2):
    @functools.partial(
        pl.kernel,
        out_type=jax.ShapeDtypeStruct((XROWS, HID), _f32),
        mesh=_scmesh(),
        scratch_types=[
            pltpu.VMEM((TPW,), jnp.int32),
            pltpu.VMEM((TPW,), jnp.int32),
            pltpu.VMEM((TPW, HID), _f32),
            pltpu.SemaphoreType.DMA,
            pltpu.SemaphoreType.DMA,
        ],
    )
    def k(tok_hbm, s1_hbm, s2_hbm, out_hbm, i1_v, i2_v, rows_v, sem1, sem2):
        wid = lax.axis_index("s") * 2 + lax.axis_index("c")
        base = wid * TPW
        c1 = pltpu.async_copy(s1_hbm.at[pl.ds(base, TPW)], i1_v, sem1)
        c2 = pltpu.async_copy(s2_hbm.at[pl.ds(base, TPW)], i2_v, sem2)
        pltpu.sync_copy(tok_hbm.at[pl.ds(base, TPW)], rows_v)
        c1.wait()
        c2.wait()
        s1 = pltpu.async_copy(rows_v, out_hbm.at[i1_v], sem1)
        s2 = pltpu.async_copy(rows_v, out_hbm.at[i2_v], sem2)
        s1.wait()
        s2.wait()

    return k(tok, slot1, slot2)


def _sc_combine(eo, slot1, slot2):
    @functools.partial(
        pl.kernel,
        out_type=(jax.ShapeDtypeStruct((NP, HID), _f32),
                  jax.ShapeDtypeStruct((NP, HID), _f32)),
        mesh=_scmesh(),
        scratch_types=[
            pltpu.VMEM((TPW,), jnp.int32),
            pltpu.VMEM((TPW,), jnp.int32),
            pltpu.VMEM((TPW, HID), _f32),
            pltpu.VMEM((TPW, HID), _f32),
            pltpu.SemaphoreType.DMA,
            pltpu.SemaphoreType.DMA,
        ],
    )
    def k(eo_hbm, s1_hbm, s2_hbm, r1_hbm, r2_hbm, i1_v, i2_v, r1_v, r2_v,
          sem1, sem2):
        wid = lax.axis_index("s") * 2 + lax.axis_index("c")
        base = wid * TPW
        c1 = pltpu.async_copy(s1_hbm.at[pl.ds(base, TPW)], i1_v, sem1)
        c2 = pltpu.async_copy(s2_hbm.at[pl.ds(base, TPW)], i2_v, sem2)
        c1.wait()
        c2.wait()
        g1 = pltpu.async_copy(eo_hbm.at[i1_v], r1_v, sem1)
        g2 = pltpu.async_copy(eo_hbm.at[i2_v], r2_v, sem2)
        g1.wait()
        w1 = pltpu.async_copy(r1_v, r1_hbm.at[pl.ds(base, TPW)], sem1)
        g2.wait()
        w2 = pltpu.async_copy(r2_v, r2_hbm.at[pl.ds(base, TPW)], sem2)
        w1.wait()
        w2.wait()

    return k(eo, slot1, slot2)


def _embed(patches, wpatch, bpatch, posb):
    return pl.pallas_call(
        _embed_body,
        grid=(1,),
        in_specs=[_full((BATCH * NPATCH, HID)), _full((HID, HID)),
                  _full((1, HID)), _full((BATCH * NPATCH, HID))],
        out_specs=_full((BATCH * NPATCH, HID)),
        out_shape=jax.ShapeDtypeStruct((BATCH * NPATCH, HID), _f32),
    )(patches, wpatch, bpatch, posb)


def _block(h, s1, b1, wq4, bq4, wk4, bk4, wv4, bv4, wo, bo, s2, b2, gw):
    return pl.pallas_call(
        _block_body,
        grid=(BATCH,),
        in_specs=[_row_spec(SP), _full((1, HID)), _full((1, HID)),
                  _full((HEADS, HID, DKV)), _full((HEADS, DKV)),
                  _full((HEADS, HID, DKV)), _full((HEADS, DKV)),
                  _full((HEADS, HID, DKV)), _full((HEADS, DKV)),
                  _full((HID, HID)), _full((1, HID)),
                  _full((1, HID)), _full((1, HID)),
                  _full((HID, E))],
        out_specs=[_row_spec(SP), _row_spec(SP),
                   pl.BlockSpec((SP, E), lambda i: (i, 0))],
        out_shape=[jax.ShapeDtypeStruct((NP, HID), _f32),
                   jax.ShapeDtypeStruct((NP, HID), _f32),
                   jax.ShapeDtypeStruct((NP, E), _f32)],
    )(h, s1, b1, wq4, bq4, wk4, bk4, wv4, bv4, wo, bo, s2, b2, gw)


def _router(probs):
    one = pl.BlockSpec((NP, 1), lambda: (0, 0))
    return pl.pallas_call(
        _router_body,
        grid=(),
        in_specs=[pl.BlockSpec((NP, E), lambda: (0, 0))],
        out_specs=[one, one, one, one],
        out_shape=[jax.ShapeDtypeStruct((NP, 1), jnp.int32),
                   jax.ShapeDtypeStruct((NP, 1), jnp.int32),
                   jax.ShapeDtypeStruct((NP, 1), _f32),
                   jax.ShapeDtypeStruct((NP, 1), _f32)],
    )(probs)


def _ffn(xd, w1, b1, w2, b2):
    return pl.pallas_call(
        _ffn_body,
        grid=(E,),
        in_specs=[pl.BlockSpec((CAPP, HID), lambda e: (e, 0)),
                  pl.BlockSpec((1, HID, DFF), lambda e: (e, 0, 0)),
                  pl.BlockSpec((1, 1, DFF), lambda e: (e, 0, 0)),
                  pl.BlockSpec((1, DFF, HID), lambda e: (e, 0, 0)),
                  pl.BlockSpec((1, 1, HID), lambda e: (e, 0, 0))],
        out_specs=pl.BlockSpec((CAPP, HID), lambda e: (e, 0)),
        out_shape=jax.ShapeDtypeStruct((XROWS, HID), _f32),
    )(xd, w1, b1, w2, b2)


def _combine(h2, r1, r2, slot1, slot2, g1, g2):
    one = pl.BlockSpec((SP, 1), lambda i: (i, 0))
    return pl.pallas_call(
        _combine_body,
        grid=(BATCH,),
        in_specs=[_row_spec(SP), _row_spec(SP), _row_spec(SP),
                  one, one, one, one],
        out_specs=_row_spec(SP),
        out_shape=jax.ShapeDtypeStruct((NP, HID), _f32),
    )(h2, r1, r2, slot1, slot2, g1, g2)


def _pool(h, s, b):
    return pl.pallas_call(
        _pool_body,
        grid=(BATCH,),
        in_specs=[_row_spec(SP), _full((1, HID)), _full((1, HID))],
        out_specs=pl.BlockSpec((1, 1, HID), lambda i: (i, 0, 0)),
        out_shape=jax.ShapeDtypeStruct((BATCH, 1, HID), _f32),
    )(h, s, b)


def _head(pooled, wc, bc):
    return pl.pallas_call(
        _head_body,
        grid=(1,),
        in_specs=[_full((BATCH, HID)), _full((HID, NCLS)), _full((1, NCLS))],
        out_specs=_full((BATCH, NCLS)),
        out_shape=jax.ShapeDtypeStruct((BATCH, NCLS), _f32),
    )(pooled, wc, bc)


def kernel(x, Wpatch, bpatch, cls_tok, pos_emb, ln1_s, ln1_b, ln2_s, ln2_b,
           Wq, bq, Wk, bk, Wv, bv, Wo, bo, gate_w, W1, b1, W2, b2,
           lnf_s, lnf_b, Wc, bc):
    patches = x.reshape(BATCH, 3, GRID, PS, GRID, PS)
    patches = patches.transpose(0, 2, 4, 1, 3, 5).reshape(BATCH * NPATCH, HID)
    posb = jnp.broadcast_to(pos_emb[:, 1:SEQ], (BATCH, NPATCH, HID))
    emb = _embed(patches, Wpatch, bpatch.reshape(1, HID),
                 posb.reshape(BATCH * NPATCH, HID))
    cls_row = jnp.broadcast_to(cls_tok + pos_emb[:, :1], (BATCH, 1, HID))
    h = jnp.concatenate(
        [cls_row, emb.reshape(BATCH, NPATCH, HID),
         jnp.zeros((BATCH, SP - SEQ, HID), _f32)], axis=1).reshape(NP, HID)

    wq4 = Wq.reshape(HID, HEADS, DKV).transpose(1, 0, 2)
    wk4 = Wk.reshape(HID, HEADS, DKV).transpose(1, 0, 2)
    wv4 = Wv.reshape(HID, HEADS, DKV).transpose(1, 0, 2)
    bq4 = bq.reshape(HEADS, DKV)
    bk4 = bk.reshape(HEADS, DKV)
    bv4 = bv.reshape(HEADS, DKV)

    for i in range(DEPTH):
        h2, hn2, probs = _block(h, ln1_s[i].reshape(1, HID),
                                ln1_b[i].reshape(1, HID),
                                wq4, bq4, wk4, bk4, wv4, bv4,
                                Wo, bo.reshape(1, HID),
                                ln2_s[i].reshape(1, HID),
                                ln2_b[i].reshape(1, HID), gate_w)
        slot1, slot2, g1, g2 = _router(probs)
        s1f = slot1.reshape(NP)
        s2f = slot2.reshape(NP)
        xd = _sc_dispatch(hn2, s1f, s2f)
        eo = _ffn(xd, W1, b1.reshape(E, 1, DFF), W2, b2.reshape(E, 1, HID))
        r1, r2 = _sc_combine(eo, s1f, s2f)
        h = _combine(h2, r1, r2, slot1, slot2, g1, g2)

    pooled = _pool(h, lnf_s.reshape(1, HID), lnf_b.reshape(1, HID))
    return _head(pooled.reshape(BATCH, HID), Wc, bc.reshape(1, NCLS))
